# baseline (speedup 1.0000x reference)
import functools
import math

import jax
import jax.numpy as jnp
from jax import lax
from jax.experimental import pallas as pl
from jax.experimental.pallas import tpu as pltpu
from jax.experimental.pallas import tpu_sc as plsc

D_MODEL = 64
SCALE = math.sqrt(D_MODEL)

NC = 2
NS = 16
NW = NC * NS
NBUF = 4
SPLIT = (120, 80)


def _make_kernel(R, S):
    assert R % (NW * NBUF) == 0 and sum(SPLIT) == S
    rpw = R // NW
    ngrp = rpw // NBUF
    mesh = plsc.VectorSubcoreMesh(
        core_axis_name="c", subcore_axis_name="s", num_cores=NC,
        num_subcores=NS)

    @functools.partial(
        pl.kernel,
        out_type=jax.ShapeDtypeStruct((R, S, D_MODEL), jnp.float32),
        mesh=mesh,
        scratch_types=[
            pltpu.VMEM((rpw, S), jnp.int32),
            pltpu.VMEM((NBUF, S, D_MODEL), jnp.float32),
            pltpu.VMEM((NBUF, S, D_MODEL), jnp.float32),
            pltpu.SemaphoreType.DMA((NBUF,)),
            pltpu.SemaphoreType.DMA((NBUF,)),
        ],
        compiler_params=pltpu.CompilerParams(use_tc_tiling_on_sc=False),
    )
    def emb_kernel(idx_hbm, table_hbm, out_hbm, idx_v, in_v, sc_v,
                   gsem, osem):
        wid = lax.axis_index("s") * NC + lax.axis_index("c")
        row0 = wid * rpw
        pltpu.sync_copy(idx_hbm.at[pl.ds(row0, rpw)], idx_v)

        def fire_gather(r, b):
            j0 = 0
            for w in SPLIT:
                pltpu.async_copy(
                    table_hbm.at[idx_v.at[r, pl.ds(j0, w)]],
                    in_v.at[b, pl.ds(j0, w)], gsem.at[b])
                j0 += w

        def wait_gather(b):
            for w in SPLIT:
                pltpu.make_async_copy(
                    table_hbm.at[idx_v.at[0, pl.ds(0, w)]],
                    in_v.at[b, pl.ds(0, w)], gsem.at[b]).wait()

        def fire_out(r, b):
            pltpu.async_copy(sc_v.at[b], out_hbm.at[row0 + r], osem.at[b])

        def wait_out(b):
            pltpu.make_async_copy(sc_v.at[b], out_hbm.at[0],
                                  osem.at[b]).wait()

        def scale(b):
            @plsc.parallel_loop(0, S, unroll=4)
            def srow(r):
                for c in range(D_MODEL // 16):
                    sl = (b, r, pl.ds(c * 16, 16))
                    sc_v[sl] = in_v[sl] * SCALE

        for b in range(NBUF):
            fire_gather(b, b)

        for b in range(NBUF):
            wait_gather(b)
            scale(b)
            fire_out(b, b)
            fire_gather(NBUF + b, b)

        @pl.loop(1, ngrp)
        def grp(g):
            r0 = g * NBUF
            for b in range(NBUF):
                wait_gather(b)
                wait_out(b)
                scale(b)
                fire_out(r0 + b, b)

                @pl.when(g < ngrp - 1)
                def _():
                    fire_gather(r0 + NBUF + b, b)

        for b in range(NBUF):
            wait_out(b)

    return emb_kernel


def kernel(x, table):
    R, S = x.shape
    return _make_kernel(R, S)(x, table)


# speedup vs baseline: 1.1488x; 1.1488x over previous
import functools
import math

import jax
import jax.numpy as jnp
from jax import lax
from jax.experimental import pallas as pl
from jax.experimental.pallas import tpu as pltpu
from jax.experimental.pallas import tpu_sc as plsc

D_MODEL = 64
SCALE = math.sqrt(D_MODEL)

NC = 2
NS = 16
NW = NC * NS
SB = 64
NBUF = 4


def _make_kernel(B):
    assert B % (NW * SB * NBUF) == 0
    npw = B // NW
    nstep = npw // SB
    ngrp = nstep // NBUF
    mesh = plsc.VectorSubcoreMesh(
        core_axis_name="c", subcore_axis_name="s", num_cores=NC,
        num_subcores=NS)

    @functools.partial(
        pl.kernel,
        out_type=jax.ShapeDtypeStruct((B, D_MODEL), jnp.float32),
        mesh=mesh,
        scratch_types=[
            pltpu.VMEM((npw + 16,), jnp.int32),
            pltpu.VMEM((NBUF, SB), jnp.int32),
            pltpu.VMEM((NBUF, SB, 2 * D_MODEL), jnp.float32),
            pltpu.VMEM((NBUF, SB, D_MODEL), jnp.float32),
            pltpu.SemaphoreType.DMA((NBUF,)),
            pltpu.SemaphoreType.DMA((NBUF,)),
        ],
        compiler_params=pltpu.CompilerParams(use_tc_tiling_on_sc=True),
    )
    def emb_kernel(idx_hbm, table_hbm, out_hbm, idx_v, rv_v, in_v, sc_v,
                   gsem, osem):
        wid = lax.axis_index("s") * NC + lax.axis_index("c")
        base = wid * npw
        pltpu.sync_copy(idx_hbm.at[pl.ds(base, npw)], idx_v.at[pl.ds(0, npw)])

        def fire_gather(s, b):
            for c in range(SB // 16):
                rv_v[b, pl.ds(c * 16, 16)] = lax.shift_right_logical(
                    idx_v[pl.ds(s * SB + c * 16, 16)], 1)
            pltpu.async_copy(table_hbm.at[rv_v.at[b]], in_v.at[b],
                             gsem.at[b])

        def wait_gather(b):
            pltpu.make_async_copy(table_hbm.at[rv_v.at[b]], in_v.at[b],
                                  gsem.at[b]).wait()

        def fire_out(s, b):
            pltpu.async_copy(sc_v.at[b], out_hbm.at[pl.ds(base + s * SB, SB)],
                             osem.at[b])

        def wait_out(b):
            pltpu.make_async_copy(sc_v.at[b], out_hbm.at[pl.ds(base, SB)],
                                  osem.at[b]).wait()

        def scale(s, b):
            @plsc.parallel_loop(0, SB, unroll=4)
            def srow(k):
                v = idx_v[pl.ds(s * SB + k, 16)]
                off = (v[0] & 1) * D_MODEL
                for c in range(D_MODEL // 16):
                    sc_v[b, k, pl.ds(c * 16, 16)] = (
                        in_v[b, k, pl.ds(off + c * 16, 16)] * SCALE)

        for b in range(NBUF):
            fire_gather(b, b)

        for b in range(NBUF):
            wait_gather(b)
            scale(b, b)
            fire_out(b, b)
            fire_gather(NBUF + b, b)

        @pl.loop(1, ngrp)
        def grp(g):
            s0 = g * NBUF
            for b in range(NBUF):
                wait_gather(b)
                wait_out(b)
                scale(s0 + b, b)
                fire_out(s0 + b, b)

                @pl.when(g < ngrp - 1)
                def _():
                    fire_gather(s0 + NBUF + b, b)

        for b in range(NBUF):
            wait_out(b)

    return emb_kernel


def kernel(x, table):
    B = x.size
    xf = x.reshape(B)
    table2 = table.reshape(table.shape[0] // 2, 2 * D_MODEL)
    out = _make_kernel(B)(xf, table2)
    return out.reshape(*x.shape, D_MODEL)
